# Initial kernel scaffold; baseline (speedup 1.0000x reference)
import jax
import jax.numpy as jnp
from jax.experimental import pallas as pl


def kernel(x, pos, edge_index, batch, emb_W, emb_b, mlp_W1, mlp_b1, mlp_W2, mlp_b2, cf_lin1_W, cf_lin2_W, cf_lin2_b, int_lin_W, int_lin_b, lin1_W, lin1_b, lin2_W, lin2_b):
    raise NotImplementedError("write your pallas kernel here")


# speedup vs baseline: 2.3265x; 2.3265x over previous
import functools

import numpy as np
import jax
import jax.numpy as jnp
from jax import lax
from jax.experimental import pallas as pl
from jax.experimental.pallas import tpu as pltpu
from jax.experimental.pallas import tpu_sc as plsc

N_NODES = 10000
N_EDGES = 320000
N_ATOMS = 21
N_CLASSES = 97
HID = 128
NG = 50
NL = 6
CUTOFF = 10.0

NC = 2
NS = 16
NW = NC * NS
CB = 128
NCHUNK = N_EDGES // CB
STRIPE = 624
TAIL = N_NODES - NS * STRIPE

_OFF = np.linspace(0.0, CUTOFF, NG).astype(np.float32)
_COEFF = np.float32(-0.5) / (_OFF[1] - _OFF[0]) ** 2
_LN2 = np.float32(np.log(2.0))

def _sc_mesh():
    return plsc.VectorSubcoreMesh(core_axis_name="c", subcore_axis_name="s",
                                  num_cores=NC, num_subcores=NS)


def _dpos_body(pos_hbm, row_hbm, col_hbm, out_hbm, idx_r, idx_c, pr_v, pc_v, dd_v, sem):
    c = lax.axis_index("c")
    s = lax.axis_index("s")
    w = s * NC + c
    nk = (NCHUNK - w + NW - 1) // NW

    def chunk(i, carry):
        base = (w + i * NW) * CB
        pltpu.sync_copy(row_hbm.at[pl.ds(base, CB)], idx_r)
        pltpu.sync_copy(col_hbm.at[pl.ds(base, CB)], idx_c)
        pltpu.async_copy(pos_hbm.at[idx_r], pr_v, sem).wait()
        pltpu.async_copy(pos_hbm.at[idx_c], pc_v, sem).wait()

        def rowfn(r, cc):
            dd_v[r, :] = pr_v[r, 0:16] - pc_v[r, 0:16]
            return cc

        lax.fori_loop(0, CB, rowfn, 0)
        pltpu.sync_copy(dd_v, out_hbm.at[pl.ds(base, CB)])
        return carry

    lax.fori_loop(0, nk, chunk, 0)


@jax.jit
def _dpos_call(pos128, row, col):
    kfn = pl.kernel(
        _dpos_body,
        out_type=jax.ShapeDtypeStruct((N_EDGES, 16), jnp.float32),
        mesh=_sc_mesh(),
        scratch_types=[
            pltpu.VMEM((CB,), jnp.int32),
            pltpu.VMEM((CB,), jnp.int32),
            pltpu.VMEM((CB, HID), jnp.float32),
            pltpu.VMEM((CB, HID), jnp.float32),
            pltpu.VMEM((CB, 16), jnp.float32),
            pltpu.SemaphoreType.DMA,
        ],
    )
    return kfn(pos128, row, col)


def _mp_body(l, xh_hbm, wf_hbm, row_hbm, col_hbm, zero_hbm, out_hbm,
             idx_r, idx_c, rows_v, wf_v, agg_sh, sem):
    c = lax.axis_index("c")
    s = lax.axis_index("s")
    w = s * NC + c
    zbase = s * STRIPE
    pltpu.sync_copy(zero_hbm.at[pl.ds(zbase, STRIPE)],
                    agg_sh.at[pl.ds(zbase, STRIPE)])

    @pl.when(s == NS - 1)
    def _():
        pltpu.sync_copy(zero_hbm.at[pl.ds(NS * STRIPE, TAIL)],
                        agg_sh.at[pl.ds(NS * STRIPE, TAIL)])

    plsc.subcore_barrier()

    nk = (NCHUNK - w + NW - 1) // NW

    def chunk(i, carry):
        base = (w + i * NW) * CB
        pltpu.sync_copy(row_hbm.at[pl.ds(base, CB)], idx_r)
        pltpu.sync_copy(col_hbm.at[pl.ds(base, CB)], idx_c)
        pltpu.async_copy(xh_hbm.at[idx_r], rows_v, sem).wait()
        pltpu.sync_copy(wf_hbm.at[l, pl.ds(base, CB)], wf_v)

        def rowfn(r, cc):
            for j in range(HID // 16):
                sl = pl.ds(j * 16, 16)
                rows_v[r, sl] = rows_v[r, sl] * wf_v[r, sl]
            return cc

        lax.fori_loop(0, CB, rowfn, 0)
        pltpu.sync_copy(rows_v, agg_sh.at[idx_c], add=True)
        return carry

    lax.fori_loop(0, nk, chunk, 0)
    plsc.subcore_barrier()
    obase = c * N_NODES + s * STRIPE
    pltpu.sync_copy(agg_sh.at[pl.ds(s * STRIPE, STRIPE)],
                    out_hbm.at[pl.ds(obase, STRIPE)])

    @pl.when(s == NS - 1)
    def _():
        pltpu.sync_copy(agg_sh.at[pl.ds(NS * STRIPE, TAIL)],
                        out_hbm.at[pl.ds(c * N_NODES + NS * STRIPE, TAIL)])


@functools.partial(jax.jit, static_argnums=(0,))
def _mp_call(l, xh, wf_all, row, col, zero):
    kfn = pl.kernel(
        functools.partial(_mp_body, l),
        out_type=jax.ShapeDtypeStruct((2 * N_NODES, HID), jnp.float32),
        mesh=_sc_mesh(),
        scratch_types=[
            pltpu.VMEM((CB,), jnp.int32),
            pltpu.VMEM((CB,), jnp.int32),
            pltpu.VMEM((CB, HID), jnp.float32),
            pltpu.VMEM((CB, HID), jnp.float32),
            pltpu.VMEM_SHARED((N_NODES, HID), jnp.float32),
            pltpu.SemaphoreType.DMA,
        ],
    )
    return kfn(xh, wf_all, row, col, zero)


BE = 512


def _wf_body(dpos_ref, w1_ref, b1_ref, w2_ref, b2_ref, out_ref):
    dp = dpos_ref[...]
    d2 = jnp.sum(dp * dp, axis=1)
    wgt = jnp.sqrt(d2 + np.float32(1e-12))
    cutc = np.float32(0.5) * (jnp.cos(wgt * np.float32(np.pi) / np.float32(CUTOFF))
                              + np.float32(1.0))
    off = (lax.broadcasted_iota(jnp.int32, (1, NG), 1).astype(jnp.float32)
           * np.float32(CUTOFF / (NG - 1)))
    diff = wgt[:, None] - off
    ea = jnp.exp(_COEFF * diff * diff)
    for l in range(NL):
        t = jnp.dot(ea, w1_ref[l], preferred_element_type=jnp.float32) + b1_ref[l]
        t = jax.nn.softplus(t) - _LN2
        wf = jnp.dot(t, w2_ref[l], preferred_element_type=jnp.float32) + b2_ref[l]
        out_ref[l] = wf * cutc[:, None]


@jax.jit
def _wf_call(dpos, mlp_W1, mlp_b1, mlp_W2, mlp_b2):
    grid = (N_EDGES // BE,)
    return pl.pallas_call(
        _wf_body,
        grid=grid,
        in_specs=[
            pl.BlockSpec((BE, 16), lambda i: (i, 0)),
            pl.BlockSpec((NL, NG, HID), lambda i: (0, 0, 0)),
            pl.BlockSpec((NL, HID), lambda i: (0, 0)),
            pl.BlockSpec((NL, HID, HID), lambda i: (0, 0, 0)),
            pl.BlockSpec((NL, HID), lambda i: (0, 0)),
        ],
        out_specs=pl.BlockSpec((NL, BE, HID), lambda i: (0, i, 0)),
        out_shape=jax.ShapeDtypeStruct((NL, N_EDGES, HID), jnp.float32),
    )(dpos, mlp_W1, mlp_b1, mlp_W2, mlp_b2)


BN = 1000


def _pre_body(x_ref, ew_ref, eb_ref, cf1_ref, h_ref, xh_ref):
    h0 = jnp.dot(x_ref[...], ew_ref[...], preferred_element_type=jnp.float32) + eb_ref[...]
    h_ref[...] = h0
    xh_ref[...] = jnp.dot(h0, cf1_ref[0], preferred_element_type=jnp.float32)


@jax.jit
def _pre_call(x, emb_W, emb_b2d, cf_lin1_W):
    grid = (N_NODES // BN,)
    return pl.pallas_call(
        _pre_body,
        grid=grid,
        in_specs=[
            pl.BlockSpec((BN, N_ATOMS), lambda i: (i, 0)),
            pl.BlockSpec((N_ATOMS, HID), lambda i: (0, 0)),
            pl.BlockSpec((1, HID), lambda i: (0, 0)),
            pl.BlockSpec((NL, HID, HID), lambda i: (0, 0, 0)),
        ],
        out_specs=[
            pl.BlockSpec((BN, HID), lambda i: (i, 0)),
            pl.BlockSpec((BN, HID), lambda i: (i, 0)),
        ],
        out_shape=[
            jax.ShapeDtypeStruct((N_NODES, HID), jnp.float32),
            jax.ShapeDtypeStruct((N_NODES, HID), jnp.float32),
        ],
    )(x, emb_W, emb_b2d, cf_lin1_W)


def _layer_body(l, a0_ref, a1_ref, h_ref, cf2w_ref, cf2b_ref, intw_ref, intb_ref,
                cf1_ref, hout_ref, xhout_ref):
    agg = a0_ref[...] + a1_ref[...]
    t = jnp.dot(agg, cf2w_ref[l], preferred_element_type=jnp.float32) + cf2b_ref[l]
    t = jax.nn.softplus(t) - _LN2
    t = jnp.dot(t, intw_ref[l], preferred_element_type=jnp.float32) + intb_ref[l]
    hn = h_ref[...] + t
    hout_ref[...] = hn
    xhout_ref[...] = jnp.dot(hn, cf1_ref[(l + 1) % NL], preferred_element_type=jnp.float32)


@functools.partial(jax.jit, static_argnums=(0,))
def _layer_call(l, agg2, h, cf_lin2_W, cf_lin2_b, int_lin_W, int_lin_b, cf_lin1_W):
    grid = (N_NODES // BN,)
    nb = N_NODES // BN
    return pl.pallas_call(
        functools.partial(_layer_body, l),
        grid=grid,
        in_specs=[
            pl.BlockSpec((BN, HID), lambda i: (i, 0)),
            pl.BlockSpec((BN, HID), lambda i, _nb=nb: (i + _nb, 0)),
            pl.BlockSpec((BN, HID), lambda i: (i, 0)),
            pl.BlockSpec((NL, HID, HID), lambda i: (0, 0, 0)),
            pl.BlockSpec((NL, HID), lambda i: (0, 0)),
            pl.BlockSpec((NL, HID, HID), lambda i: (0, 0, 0)),
            pl.BlockSpec((NL, HID), lambda i: (0, 0)),
            pl.BlockSpec((NL, HID, HID), lambda i: (0, 0, 0)),
        ],
        out_specs=[
            pl.BlockSpec((BN, HID), lambda i: (i, 0)),
            pl.BlockSpec((BN, HID), lambda i: (i, 0)),
        ],
        out_shape=[
            jax.ShapeDtypeStruct((N_NODES, HID), jnp.float32),
            jax.ShapeDtypeStruct((N_NODES, HID), jnp.float32),
        ],
    )(agg2, agg2, h, cf_lin2_W, cf_lin2_b, int_lin_W, int_lin_b, cf_lin1_W)


def _head_body(h_ref, l1w_ref, l1b_ref, l2w_ref, l2b_ref, ge_ref, out_ref, acc_ref):
    i = pl.program_id(0)
    hf = jnp.maximum(
        jnp.dot(h_ref[...], l1w_ref[...], preferred_element_type=jnp.float32)
        + l1b_ref[...], np.float32(0.0))
    part = jnp.sum(hf, axis=0, keepdims=True)

    @pl.when(i == 0)
    def _():
        acc_ref[...] = part

    @pl.when(i > 0)
    def _():
        acc_ref[...] = acc_ref[...] + part

    @pl.when(i == (N_NODES // BN) - 1)
    def _():
        ge = acc_ref[...] / np.float32(N_NODES)
        ge_ref[...] = ge
        h2 = jnp.maximum(
            jnp.dot(ge, l1w_ref[...], preferred_element_type=jnp.float32)
            + l1b_ref[...], np.float32(0.0))
        out_ref[...] = (jnp.dot(h2, l2w_ref[...], preferred_element_type=jnp.float32)
                        + l2b_ref[...])


@jax.jit
def _head_call(h, lin1_W, lin1_b2d, lin2_W, lin2_b2d):
    grid = (N_NODES // BN,)
    return pl.pallas_call(
        _head_body,
        grid=grid,
        in_specs=[
            pl.BlockSpec((BN, HID), lambda i: (i, 0)),
            pl.BlockSpec((HID, HID), lambda i: (0, 0)),
            pl.BlockSpec((1, HID), lambda i: (0, 0)),
            pl.BlockSpec((HID, N_CLASSES), lambda i: (0, 0)),
            pl.BlockSpec((1, N_CLASSES), lambda i: (0, 0)),
        ],
        out_specs=[
            pl.BlockSpec((1, HID), lambda i: (0, 0)),
            pl.BlockSpec((1, N_CLASSES), lambda i: (0, 0)),
        ],
        out_shape=[
            jax.ShapeDtypeStruct((1, HID), jnp.float32),
            jax.ShapeDtypeStruct((1, N_CLASSES), jnp.float32),
        ],
        scratch_shapes=[pltpu.VMEM((1, HID), jnp.float32)],
    )(h, lin1_W, lin1_b2d, lin2_W, lin2_b2d)


def kernel(x, pos, edge_index, batch, emb_W, emb_b, mlp_W1, mlp_b1, mlp_W2, mlp_b2,
           cf_lin1_W, cf_lin2_W, cf_lin2_b, int_lin_W, int_lin_b,
           lin1_W, lin1_b, lin2_W, lin2_b):
    row = edge_index[0].astype(jnp.int32)
    col = edge_index[1].astype(jnp.int32)
    pos128 = jnp.zeros((N_NODES, HID), jnp.float32).at[:, :3].set(pos)

    dpos = _dpos_call(pos128, row, col)
    wf_all = _wf_call(dpos, mlp_W1, mlp_b1, mlp_W2, mlp_b2)
    h, xh = _pre_call(x, emb_W, emb_b.reshape(1, HID), cf_lin1_W)

    zero = jnp.zeros((N_NODES, HID), jnp.float32)
    for l in range(NL):
        agg2 = _mp_call(l, xh, wf_all, row, col, zero)
        h, xh = _layer_call(l, agg2, h, cf_lin2_W, cf_lin2_b, int_lin_W,
                            int_lin_b, cf_lin1_W)

    ge, out = _head_call(h, lin1_W, lin1_b.reshape(1, HID), lin2_W,
                         lin2_b.reshape(1, N_CLASSES))
    return (ge, out)


# speedup vs baseline: 2.6794x; 1.1516x over previous
import functools

import numpy as np
import jax
import jax.numpy as jnp
from jax import lax
from jax.experimental import pallas as pl
from jax.experimental.pallas import tpu as pltpu
from jax.experimental.pallas import tpu_sc as plsc

N_NODES = 10000
N_EDGES = 320000
N_ATOMS = 21
N_CLASSES = 97
HID = 128
NG = 50
NL = 6
CUTOFF = 10.0

NC = 2
NS = 16
NW = NC * NS
CB = 128
NCHUNK = N_EDGES // CB
STRIPE = 624
TAIL = N_NODES - NS * STRIPE

_OFF = np.linspace(0.0, CUTOFF, NG).astype(np.float32)
_COEFF = np.float32(-0.5) / (_OFF[1] - _OFF[0]) ** 2
_LN2 = np.float32(np.log(2.0))

def _ssp(v):
    return (jnp.maximum(v, np.float32(0.0))
            + jnp.log(np.float32(1.0) + jnp.exp(-jnp.abs(v))) - _LN2)


def _sc_mesh():
    return plsc.VectorSubcoreMesh(core_axis_name="c", subcore_axis_name="s",
                                  num_cores=NC, num_subcores=NS)


def _dpos_body(pos_hbm, row_hbm, col_hbm, out_hbm, idx_r, idx_c, pr_v, pc_v, dd_v, sem):
    c = lax.axis_index("c")
    s = lax.axis_index("s")
    w = s * NC + c
    nk = (NCHUNK - w + NW - 1) // NW

    def chunk(i, carry):
        base = (w + i * NW) * CB
        pltpu.sync_copy(row_hbm.at[pl.ds(base, CB)], idx_r)
        pltpu.sync_copy(col_hbm.at[pl.ds(base, CB)], idx_c)
        pltpu.async_copy(pos_hbm.at[idx_r], pr_v, sem).wait()
        pltpu.async_copy(pos_hbm.at[idx_c], pc_v, sem).wait()

        def rowfn(r, cc):
            dd_v[r, :] = pr_v[r, 0:16] - pc_v[r, 0:16]
            return cc

        lax.fori_loop(0, CB, rowfn, 0)
        pltpu.sync_copy(dd_v, out_hbm.at[pl.ds(base, CB)])
        return carry

    lax.fori_loop(0, nk, chunk, 0)


@jax.jit
def _dpos_call(pos128, row, col):
    kfn = pl.kernel(
        _dpos_body,
        out_type=jax.ShapeDtypeStruct((N_EDGES, 16), jnp.float32),
        mesh=_sc_mesh(),
        scratch_types=[
            pltpu.VMEM((CB,), jnp.int32),
            pltpu.VMEM((CB,), jnp.int32),
            pltpu.VMEM((CB, HID), jnp.float32),
            pltpu.VMEM((CB, HID), jnp.float32),
            pltpu.VMEM((CB, 16), jnp.float32),
            pltpu.SemaphoreType.DMA,
        ],
    )
    return kfn(pos128, row, col)


def _mp_body(l, xh_hbm, wf_hbm, row_hbm, col_hbm, zero_hbm, out_hbm,
             idx_r, idx_c, rows_v, wf_v, agg_sh, sem):
    c = lax.axis_index("c")
    s = lax.axis_index("s")
    w = s * NC + c
    zbase = s * STRIPE
    pltpu.sync_copy(zero_hbm.at[pl.ds(zbase, STRIPE)],
                    agg_sh.at[pl.ds(zbase, STRIPE)])

    @pl.when(s == NS - 1)
    def _():
        pltpu.sync_copy(zero_hbm.at[pl.ds(NS * STRIPE, TAIL)],
                        agg_sh.at[pl.ds(NS * STRIPE, TAIL)])

    plsc.subcore_barrier()

    nk = (NCHUNK - w + NW - 1) // NW

    def chunk(i, carry):
        base = (w + i * NW) * CB
        pltpu.sync_copy(row_hbm.at[pl.ds(base, CB)], idx_r)
        pltpu.sync_copy(col_hbm.at[pl.ds(base, CB)], idx_c)
        pltpu.async_copy(xh_hbm.at[idx_r], rows_v, sem).wait()
        pltpu.sync_copy(wf_hbm.at[l, pl.ds(base, CB)], wf_v)

        def rowfn(r, cc):
            for j in range(HID // 16):
                sl = pl.ds(j * 16, 16)
                rows_v[r, sl] = rows_v[r, sl] * wf_v[r, sl]
            return cc

        lax.fori_loop(0, CB, rowfn, 0)
        pltpu.sync_copy(rows_v, agg_sh.at[idx_c], add=True)
        return carry

    lax.fori_loop(0, nk, chunk, 0)
    plsc.subcore_barrier()
    obase = c * N_NODES + s * STRIPE
    pltpu.sync_copy(agg_sh.at[pl.ds(s * STRIPE, STRIPE)],
                    out_hbm.at[pl.ds(obase, STRIPE)])

    @pl.when(s == NS - 1)
    def _():
        pltpu.sync_copy(agg_sh.at[pl.ds(NS * STRIPE, TAIL)],
                        out_hbm.at[pl.ds(c * N_NODES + NS * STRIPE, TAIL)])


@functools.partial(jax.jit, static_argnums=(0,))
def _mp_call(l, xh, wf_all, row, col, zero):
    kfn = pl.kernel(
        functools.partial(_mp_body, l),
        out_type=jax.ShapeDtypeStruct((2 * N_NODES, HID), jnp.float32),
        mesh=_sc_mesh(),
        scratch_types=[
            pltpu.VMEM((CB,), jnp.int32),
            pltpu.VMEM((CB,), jnp.int32),
            pltpu.VMEM((CB, HID), jnp.float32),
            pltpu.VMEM((CB, HID), jnp.float32),
            pltpu.VMEM_SHARED((N_NODES, HID), jnp.float32),
            pltpu.SemaphoreType.DMA,
        ],
    )
    return kfn(xh, wf_all, row, col, zero)


BE = 512


def _wf_body(dpos_ref, w1_ref, b1_ref, w2_ref, b2_ref, out_ref):
    dp = dpos_ref[...]
    d2 = jnp.sum(dp * dp, axis=1)
    wgt = jnp.sqrt(d2 + np.float32(1e-12))
    xx = wgt * np.float32(np.pi / CUTOFF)
    k = jnp.floor(xx * np.float32(1.0 / np.pi) + np.float32(0.5))
    r = xx - k * np.float32(np.pi)
    r2 = r * r
    cpoly = np.float32(-2.605210867e-07)
    for coef in (2.479886187e-05, -1.388829677e-03, 4.166645418e-02,
                 -4.999999389e-01, 9.999999724e-01):
        cpoly = cpoly * r2 + np.float32(coef)
    kodd = k - np.float32(2.0) * jnp.floor(k * np.float32(0.5))
    sgn = np.float32(1.0) - np.float32(2.0) * kodd
    cutc = np.float32(0.5) * (sgn * cpoly + np.float32(1.0))
    off = (lax.broadcasted_iota(jnp.int32, (1, NG), 1).astype(jnp.float32)
           * np.float32(CUTOFF / (NG - 1)))
    diff = wgt[:, None] - off
    ea = jnp.exp(_COEFF * diff * diff)
    for l in range(NL):
        t = jnp.dot(ea, w1_ref[l], preferred_element_type=jnp.float32) + b1_ref[l]
        t = _ssp(t)
        wf = jnp.dot(t, w2_ref[l], preferred_element_type=jnp.float32) + b2_ref[l]
        out_ref[l] = wf * cutc[:, None]


@jax.jit
def _wf_call(dpos, mlp_W1, mlp_b1, mlp_W2, mlp_b2):
    grid = (N_EDGES // BE,)
    return pl.pallas_call(
        _wf_body,
        grid=grid,
        in_specs=[
            pl.BlockSpec((BE, 16), lambda i: (i, 0)),
            pl.BlockSpec((NL, NG, HID), lambda i: (0, 0, 0)),
            pl.BlockSpec((NL, HID), lambda i: (0, 0)),
            pl.BlockSpec((NL, HID, HID), lambda i: (0, 0, 0)),
            pl.BlockSpec((NL, HID), lambda i: (0, 0)),
        ],
        out_specs=pl.BlockSpec((NL, BE, HID), lambda i: (0, i, 0)),
        out_shape=jax.ShapeDtypeStruct((NL, N_EDGES, HID), jnp.float32),
    )(dpos, mlp_W1, mlp_b1, mlp_W2, mlp_b2)


BN = 1000


def _pre_body(x_ref, ew_ref, eb_ref, cf1_ref, h_ref, xh_ref):
    h0 = jnp.dot(x_ref[...], ew_ref[...], preferred_element_type=jnp.float32) + eb_ref[...]
    h_ref[...] = h0
    xh_ref[...] = jnp.dot(h0, cf1_ref[0], preferred_element_type=jnp.float32)


@jax.jit
def _pre_call(x, emb_W, emb_b2d, cf_lin1_W):
    grid = (N_NODES // BN,)
    return pl.pallas_call(
        _pre_body,
        grid=grid,
        in_specs=[
            pl.BlockSpec((BN, N_ATOMS), lambda i: (i, 0)),
            pl.BlockSpec((N_ATOMS, HID), lambda i: (0, 0)),
            pl.BlockSpec((1, HID), lambda i: (0, 0)),
            pl.BlockSpec((NL, HID, HID), lambda i: (0, 0, 0)),
        ],
        out_specs=[
            pl.BlockSpec((BN, HID), lambda i: (i, 0)),
            pl.BlockSpec((BN, HID), lambda i: (i, 0)),
        ],
        out_shape=[
            jax.ShapeDtypeStruct((N_NODES, HID), jnp.float32),
            jax.ShapeDtypeStruct((N_NODES, HID), jnp.float32),
        ],
    )(x, emb_W, emb_b2d, cf_lin1_W)


def _layer_body(l, a0_ref, a1_ref, h_ref, cf2w_ref, cf2b_ref, intw_ref, intb_ref,
                cf1_ref, hout_ref, xhout_ref):
    agg = a0_ref[...] + a1_ref[...]
    t = jnp.dot(agg, cf2w_ref[l], preferred_element_type=jnp.float32) + cf2b_ref[l]
    t = _ssp(t)
    t = jnp.dot(t, intw_ref[l], preferred_element_type=jnp.float32) + intb_ref[l]
    hn = h_ref[...] + t
    hout_ref[...] = hn
    xhout_ref[...] = jnp.dot(hn, cf1_ref[(l + 1) % NL], preferred_element_type=jnp.float32)


@functools.partial(jax.jit, static_argnums=(0,))
def _layer_call(l, agg2, h, cf_lin2_W, cf_lin2_b, int_lin_W, int_lin_b, cf_lin1_W):
    grid = (N_NODES // BN,)
    nb = N_NODES // BN
    return pl.pallas_call(
        functools.partial(_layer_body, l),
        grid=grid,
        in_specs=[
            pl.BlockSpec((BN, HID), lambda i: (i, 0)),
            pl.BlockSpec((BN, HID), lambda i, _nb=nb: (i + _nb, 0)),
            pl.BlockSpec((BN, HID), lambda i: (i, 0)),
            pl.BlockSpec((NL, HID, HID), lambda i: (0, 0, 0)),
            pl.BlockSpec((NL, HID), lambda i: (0, 0)),
            pl.BlockSpec((NL, HID, HID), lambda i: (0, 0, 0)),
            pl.BlockSpec((NL, HID), lambda i: (0, 0)),
            pl.BlockSpec((NL, HID, HID), lambda i: (0, 0, 0)),
        ],
        out_specs=[
            pl.BlockSpec((BN, HID), lambda i: (i, 0)),
            pl.BlockSpec((BN, HID), lambda i: (i, 0)),
        ],
        out_shape=[
            jax.ShapeDtypeStruct((N_NODES, HID), jnp.float32),
            jax.ShapeDtypeStruct((N_NODES, HID), jnp.float32),
        ],
    )(agg2, agg2, h, cf_lin2_W, cf_lin2_b, int_lin_W, int_lin_b, cf_lin1_W)


def _head_body(h_ref, l1w_ref, l1b_ref, l2w_ref, l2b_ref, ge_ref, out_ref, acc_ref):
    i = pl.program_id(0)
    hf = jnp.maximum(
        jnp.dot(h_ref[...], l1w_ref[...], preferred_element_type=jnp.float32)
        + l1b_ref[...], np.float32(0.0))
    part = jnp.sum(hf, axis=0, keepdims=True)

    @pl.when(i == 0)
    def _():
        acc_ref[...] = part

    @pl.when(i > 0)
    def _():
        acc_ref[...] = acc_ref[...] + part

    @pl.when(i == (N_NODES // BN) - 1)
    def _():
        ge = acc_ref[...] / np.float32(N_NODES)
        ge_ref[...] = ge
        h2 = jnp.maximum(
            jnp.dot(ge, l1w_ref[...], preferred_element_type=jnp.float32)
            + l1b_ref[...], np.float32(0.0))
        out_ref[...] = (jnp.dot(h2, l2w_ref[...], preferred_element_type=jnp.float32)
                        + l2b_ref[...])


@jax.jit
def _head_call(h, lin1_W, lin1_b2d, lin2_W, lin2_b2d):
    grid = (N_NODES // BN,)
    return pl.pallas_call(
        _head_body,
        grid=grid,
        in_specs=[
            pl.BlockSpec((BN, HID), lambda i: (i, 0)),
            pl.BlockSpec((HID, HID), lambda i: (0, 0)),
            pl.BlockSpec((1, HID), lambda i: (0, 0)),
            pl.BlockSpec((HID, N_CLASSES), lambda i: (0, 0)),
            pl.BlockSpec((1, N_CLASSES), lambda i: (0, 0)),
        ],
        out_specs=[
            pl.BlockSpec((1, HID), lambda i: (0, 0)),
            pl.BlockSpec((1, N_CLASSES), lambda i: (0, 0)),
        ],
        out_shape=[
            jax.ShapeDtypeStruct((1, HID), jnp.float32),
            jax.ShapeDtypeStruct((1, N_CLASSES), jnp.float32),
        ],
        scratch_shapes=[pltpu.VMEM((1, HID), jnp.float32)],
    )(h, lin1_W, lin1_b2d, lin2_W, lin2_b2d)


def kernel(x, pos, edge_index, batch, emb_W, emb_b, mlp_W1, mlp_b1, mlp_W2, mlp_b2,
           cf_lin1_W, cf_lin2_W, cf_lin2_b, int_lin_W, int_lin_b,
           lin1_W, lin1_b, lin2_W, lin2_b):
    row = edge_index[0].astype(jnp.int32)
    col = edge_index[1].astype(jnp.int32)
    pos128 = jnp.zeros((N_NODES, HID), jnp.float32).at[:, :3].set(pos)

    dpos = _dpos_call(pos128, row, col)
    wf_all = _wf_call(dpos, mlp_W1, mlp_b1, mlp_W2, mlp_b2)
    h, xh = _pre_call(x, emb_W, emb_b.reshape(1, HID), cf_lin1_W)

    zero = jnp.zeros((N_NODES, HID), jnp.float32)
    for l in range(NL):
        agg2 = _mp_call(l, xh, wf_all, row, col, zero)
        h, xh = _layer_call(l, agg2, h, cf_lin2_W, cf_lin2_b, int_lin_W,
                            int_lin_b, cf_lin1_W)

    ge, out = _head_call(h, lin1_W, lin1_b.reshape(1, HID), lin2_W,
                         lin2_b.reshape(1, N_CLASSES))
    return (ge, out)


# speedup vs baseline: 3.4502x; 1.2877x over previous
import functools

import numpy as np
import jax
import jax.numpy as jnp
from jax import lax
from jax.experimental import pallas as pl
from jax.experimental.pallas import tpu as pltpu
from jax.experimental.pallas import tpu_sc as plsc

N_NODES = 10000
N_EDGES = 320000
N_ATOMS = 21
N_CLASSES = 97
HID = 128
NG = 50
NL = 6
CUTOFF = 10.0

NC = 2
NS = 16
NW = NC * NS
CB = 64
NCHUNK = N_EDGES // CB
STRIPE = 624
TAIL = N_NODES - NS * STRIPE

_OFF = np.linspace(0.0, CUTOFF, NG).astype(np.float32)
_COEFF = np.float32(-0.5) / (_OFF[1] - _OFF[0]) ** 2
_LN2 = np.float32(np.log(2.0))

def _ssp(v):
    return (jnp.maximum(v, np.float32(0.0))
            + jnp.log(np.float32(1.0) + jnp.exp(-jnp.abs(v))) - _LN2)


def _sc_mesh():
    return plsc.VectorSubcoreMesh(core_axis_name="c", subcore_axis_name="s",
                                  num_cores=NC, num_subcores=NS)


def _dpos_body(pos_hbm, row_hbm, col_hbm, out_hbm, idx_r, idx_c, pr_v, pc_v, dd_v, sem):
    c = lax.axis_index("c")
    s = lax.axis_index("s")
    w = s * NC + c
    nk = (NCHUNK - w + NW - 1) // NW

    def chunk(i, carry):
        base = (w + i * NW) * CB
        pltpu.sync_copy(row_hbm.at[pl.ds(base, CB)], idx_r)
        pltpu.sync_copy(col_hbm.at[pl.ds(base, CB)], idx_c)
        pltpu.async_copy(pos_hbm.at[idx_r], pr_v, sem).wait()
        pltpu.async_copy(pos_hbm.at[idx_c], pc_v, sem).wait()

        def rowfn(r, cc):
            dd_v[r, :] = pr_v[r, 0:16] - pc_v[r, 0:16]
            return cc

        lax.fori_loop(0, CB, rowfn, 0)
        pltpu.sync_copy(dd_v, out_hbm.at[pl.ds(base, CB)])
        return carry

    lax.fori_loop(0, nk, chunk, 0)


@jax.jit
def _dpos_call(pos128, row, col):
    kfn = pl.kernel(
        _dpos_body,
        out_type=jax.ShapeDtypeStruct((N_EDGES, 16), jnp.float32),
        mesh=_sc_mesh(),
        scratch_types=[
            pltpu.VMEM((CB,), jnp.int32),
            pltpu.VMEM((CB,), jnp.int32),
            pltpu.VMEM((CB, HID), jnp.float32),
            pltpu.VMEM((CB, HID), jnp.float32),
            pltpu.VMEM((CB, 16), jnp.float32),
            pltpu.SemaphoreType.DMA,
        ],
    )
    return kfn(pos128, row, col)


NKFULL = 156
NPAIR = NKFULL // 3


def _mp_body(l, xh_hbm, wf_hbm, row_hbm, col_hbm, zero_hbm, out_hbm,
             idx_r, idx_c, rows, wfv, agg_sh,
             sg0, sg1, sg2, sw0, sw1, sw2, ss0, ss1, ss2):
    SG = (sg0, sg1, sg2)
    SW = (sw0, sw1, sw2)
    SS = (ss0, ss1, ss2)
    c = lax.axis_index("c")
    s = lax.axis_index("s")
    w = s * NC + c
    zbase = s * STRIPE
    pltpu.sync_copy(zero_hbm.at[pl.ds(zbase, STRIPE)],
                    agg_sh.at[pl.ds(zbase, STRIPE)])

    @pl.when(s == NS - 1)
    def _():
        pltpu.sync_copy(zero_hbm.at[pl.ds(NS * STRIPE, TAIL)],
                        agg_sh.at[pl.ds(NS * STRIPE, TAIL)])

    plsc.subcore_barrier()

    def start(t, b):
        base = (w + t * NW) * CB
        pltpu.sync_copy(row_hbm.at[pl.ds(base, CB)], idx_r.at[b])
        pltpu.sync_copy(col_hbm.at[pl.ds(base, CB)], idx_c.at[b])
        pltpu.async_copy(xh_hbm.at[idx_r.at[b]], rows.at[b], SG[b])
        pltpu.async_copy(wf_hbm.at[l, pl.ds(base, CB)], wfv.at[b], SW[b])

    def drain_scatter(b):
        pltpu.make_async_copy(rows.at[b], agg_sh.at[idx_c.at[b]], SS[b]).wait()

    def mul(b):
        def rowfn(r, cc):
            for j in range(HID // 16):
                sl = pl.ds(j * 16, 16)
                rows[b, r, sl] = rows[b, r, sl] * wfv[b, r, sl]
            return cc

        lax.fori_loop(0, CB, rowfn, 0)

    start(0, 0)
    start(1, 1)

    def block(j, r):
        t = 3 * j + r
        b2 = (r + 2) % 3
        pltpu.make_async_copy(xh_hbm.at[idx_r.at[r]], rows.at[r], SG[r]).wait()
        pltpu.make_async_copy(wf_hbm.at[l, pl.ds(0, CB)], wfv.at[r], SW[r]).wait()
        mul(r)
        pltpu.async_copy(rows.at[r], agg_sh.at[idx_c.at[r]], SS[r], add=True)
        return t, b2

    def pair(j, carry):
        t, b2 = block(j, 0)

        @pl.when(j >= 1)
        def _():
            drain_scatter(b2)

        start(t + 2, b2)

        t, b2 = block(j, 1)

        @pl.when(j < NPAIR - 1)
        def _():
            drain_scatter(b2)
            start(t + 2, b2)

        t, b2 = block(j, 2)

        @pl.when(j < NPAIR - 1)
        def _():
            drain_scatter(b2)
            start(t + 2, b2)

        return carry

    lax.fori_loop(0, NPAIR, pair, 0)
    drain_scatter(0)
    drain_scatter(1)
    drain_scatter(2)

    @pl.when(w < NCHUNK - NKFULL * NW)
    def _():
        base = (NKFULL * NW + w) * CB
        pltpu.sync_copy(row_hbm.at[pl.ds(base, CB)], idx_r.at[0])
        pltpu.sync_copy(col_hbm.at[pl.ds(base, CB)], idx_c.at[0])
        pltpu.async_copy(xh_hbm.at[idx_r.at[0]], rows.at[0], SG[0]).wait()
        pltpu.sync_copy(wf_hbm.at[l, pl.ds(base, CB)], wfv.at[0])
        mul(0)
        pltpu.sync_copy(rows.at[0], agg_sh.at[idx_c.at[0]], add=True)

    plsc.subcore_barrier()
    obase = c * N_NODES + s * STRIPE
    pltpu.sync_copy(agg_sh.at[pl.ds(s * STRIPE, STRIPE)],
                    out_hbm.at[pl.ds(obase, STRIPE)])

    @pl.when(s == NS - 1)
    def _():
        pltpu.sync_copy(agg_sh.at[pl.ds(NS * STRIPE, TAIL)],
                        out_hbm.at[pl.ds(c * N_NODES + NS * STRIPE, TAIL)])


@functools.partial(jax.jit, static_argnums=(0,))
def _mp_call(l, xh, wf_all, row, col, zero):
    kfn = pl.kernel(
        functools.partial(_mp_body, l),
        out_type=jax.ShapeDtypeStruct((2 * N_NODES, HID), jnp.float32),
        mesh=_sc_mesh(),
        scratch_types=[
            pltpu.VMEM((3, CB), jnp.int32),
            pltpu.VMEM((3, CB), jnp.int32),
            pltpu.VMEM((3, CB, HID), jnp.float32),
            pltpu.VMEM((3, CB, HID), jnp.float32),
            pltpu.VMEM_SHARED((N_NODES, HID), jnp.float32),
        ] + [pltpu.SemaphoreType.DMA] * 9,
    )
    return kfn(xh, wf_all, row, col, zero)


BE = 512


def _wf_body(dpos_ref, w1_ref, b1_ref, w2_ref, b2_ref, out_ref):
    dp = dpos_ref[...]
    d2 = jnp.sum(dp * dp, axis=1)
    wgt = jnp.sqrt(d2 + np.float32(1e-12))
    xx = wgt * np.float32(np.pi / CUTOFF)
    k = jnp.floor(xx * np.float32(1.0 / np.pi) + np.float32(0.5))
    r = xx - k * np.float32(np.pi)
    r2 = r * r
    cpoly = np.float32(-2.605210867e-07)
    for coef in (2.479886187e-05, -1.388829677e-03, 4.166645418e-02,
                 -4.999999389e-01, 9.999999724e-01):
        cpoly = cpoly * r2 + np.float32(coef)
    kodd = k - np.float32(2.0) * jnp.floor(k * np.float32(0.5))
    sgn = np.float32(1.0) - np.float32(2.0) * kodd
    cutc = np.float32(0.5) * (sgn * cpoly + np.float32(1.0))
    off = (lax.broadcasted_iota(jnp.int32, (1, NG), 1).astype(jnp.float32)
           * np.float32(CUTOFF / (NG - 1)))
    diff = wgt[:, None] - off
    ea = jnp.exp(_COEFF * diff * diff)
    for l in range(NL):
        t = jnp.dot(ea, w1_ref[l], preferred_element_type=jnp.float32) + b1_ref[l]
        t = _ssp(t)
        wf = jnp.dot(t, w2_ref[l], preferred_element_type=jnp.float32) + b2_ref[l]
        out_ref[l] = wf * cutc[:, None]


@jax.jit
def _wf_call(dpos, mlp_W1, mlp_b1, mlp_W2, mlp_b2):
    grid = (N_EDGES // BE,)
    return pl.pallas_call(
        _wf_body,
        grid=grid,
        in_specs=[
            pl.BlockSpec((BE, 16), lambda i: (i, 0)),
            pl.BlockSpec((NL, NG, HID), lambda i: (0, 0, 0)),
            pl.BlockSpec((NL, HID), lambda i: (0, 0)),
            pl.BlockSpec((NL, HID, HID), lambda i: (0, 0, 0)),
            pl.BlockSpec((NL, HID), lambda i: (0, 0)),
        ],
        out_specs=pl.BlockSpec((NL, BE, HID), lambda i: (0, i, 0)),
        out_shape=jax.ShapeDtypeStruct((NL, N_EDGES, HID), jnp.float32),
    )(dpos, mlp_W1, mlp_b1, mlp_W2, mlp_b2)


BN = 1000


def _pre_body(x_ref, ew_ref, eb_ref, cf1_ref, h_ref, xh_ref):
    h0 = jnp.dot(x_ref[...], ew_ref[...], preferred_element_type=jnp.float32) + eb_ref[...]
    h_ref[...] = h0
    xh_ref[...] = jnp.dot(h0, cf1_ref[0], preferred_element_type=jnp.float32)


@jax.jit
def _pre_call(x, emb_W, emb_b2d, cf_lin1_W):
    grid = (N_NODES // BN,)
    return pl.pallas_call(
        _pre_body,
        grid=grid,
        in_specs=[
            pl.BlockSpec((BN, N_ATOMS), lambda i: (i, 0)),
            pl.BlockSpec((N_ATOMS, HID), lambda i: (0, 0)),
            pl.BlockSpec((1, HID), lambda i: (0, 0)),
            pl.BlockSpec((NL, HID, HID), lambda i: (0, 0, 0)),
        ],
        out_specs=[
            pl.BlockSpec((BN, HID), lambda i: (i, 0)),
            pl.BlockSpec((BN, HID), lambda i: (i, 0)),
        ],
        out_shape=[
            jax.ShapeDtypeStruct((N_NODES, HID), jnp.float32),
            jax.ShapeDtypeStruct((N_NODES, HID), jnp.float32),
        ],
    )(x, emb_W, emb_b2d, cf_lin1_W)


def _layer_body(l, a0_ref, a1_ref, h_ref, cf2w_ref, cf2b_ref, intw_ref, intb_ref,
                cf1_ref, hout_ref, xhout_ref):
    agg = a0_ref[...] + a1_ref[...]
    t = jnp.dot(agg, cf2w_ref[l], preferred_element_type=jnp.float32) + cf2b_ref[l]
    t = _ssp(t)
    t = jnp.dot(t, intw_ref[l], preferred_element_type=jnp.float32) + intb_ref[l]
    hn = h_ref[...] + t
    hout_ref[...] = hn
    xhout_ref[...] = jnp.dot(hn, cf1_ref[(l + 1) % NL], preferred_element_type=jnp.float32)


@functools.partial(jax.jit, static_argnums=(0,))
def _layer_call(l, agg2, h, cf_lin2_W, cf_lin2_b, int_lin_W, int_lin_b, cf_lin1_W):
    grid = (N_NODES // BN,)
    nb = N_NODES // BN
    return pl.pallas_call(
        functools.partial(_layer_body, l),
        grid=grid,
        in_specs=[
            pl.BlockSpec((BN, HID), lambda i: (i, 0)),
            pl.BlockSpec((BN, HID), lambda i, _nb=nb: (i + _nb, 0)),
            pl.BlockSpec((BN, HID), lambda i: (i, 0)),
            pl.BlockSpec((NL, HID, HID), lambda i: (0, 0, 0)),
            pl.BlockSpec((NL, HID), lambda i: (0, 0)),
            pl.BlockSpec((NL, HID, HID), lambda i: (0, 0, 0)),
            pl.BlockSpec((NL, HID), lambda i: (0, 0)),
            pl.BlockSpec((NL, HID, HID), lambda i: (0, 0, 0)),
        ],
        out_specs=[
            pl.BlockSpec((BN, HID), lambda i: (i, 0)),
            pl.BlockSpec((BN, HID), lambda i: (i, 0)),
        ],
        out_shape=[
            jax.ShapeDtypeStruct((N_NODES, HID), jnp.float32),
            jax.ShapeDtypeStruct((N_NODES, HID), jnp.float32),
        ],
    )(agg2, agg2, h, cf_lin2_W, cf_lin2_b, int_lin_W, int_lin_b, cf_lin1_W)


def _head_body(h_ref, l1w_ref, l1b_ref, l2w_ref, l2b_ref, ge_ref, out_ref, acc_ref):
    i = pl.program_id(0)
    hf = jnp.maximum(
        jnp.dot(h_ref[...], l1w_ref[...], preferred_element_type=jnp.float32)
        + l1b_ref[...], np.float32(0.0))
    part = jnp.sum(hf, axis=0, keepdims=True)

    @pl.when(i == 0)
    def _():
        acc_ref[...] = part

    @pl.when(i > 0)
    def _():
        acc_ref[...] = acc_ref[...] + part

    @pl.when(i == (N_NODES // BN) - 1)
    def _():
        ge = acc_ref[...] / np.float32(N_NODES)
        ge_ref[...] = ge
        h2 = jnp.maximum(
            jnp.dot(ge, l1w_ref[...], preferred_element_type=jnp.float32)
            + l1b_ref[...], np.float32(0.0))
        out_ref[...] = (jnp.dot(h2, l2w_ref[...], preferred_element_type=jnp.float32)
                        + l2b_ref[...])


@jax.jit
def _head_call(h, lin1_W, lin1_b2d, lin2_W, lin2_b2d):
    grid = (N_NODES // BN,)
    return pl.pallas_call(
        _head_body,
        grid=grid,
        in_specs=[
            pl.BlockSpec((BN, HID), lambda i: (i, 0)),
            pl.BlockSpec((HID, HID), lambda i: (0, 0)),
            pl.BlockSpec((1, HID), lambda i: (0, 0)),
            pl.BlockSpec((HID, N_CLASSES), lambda i: (0, 0)),
            pl.BlockSpec((1, N_CLASSES), lambda i: (0, 0)),
        ],
        out_specs=[
            pl.BlockSpec((1, HID), lambda i: (0, 0)),
            pl.BlockSpec((1, N_CLASSES), lambda i: (0, 0)),
        ],
        out_shape=[
            jax.ShapeDtypeStruct((1, HID), jnp.float32),
            jax.ShapeDtypeStruct((1, N_CLASSES), jnp.float32),
        ],
        scratch_shapes=[pltpu.VMEM((1, HID), jnp.float32)],
    )(h, lin1_W, lin1_b2d, lin2_W, lin2_b2d)


def kernel(x, pos, edge_index, batch, emb_W, emb_b, mlp_W1, mlp_b1, mlp_W2, mlp_b2,
           cf_lin1_W, cf_lin2_W, cf_lin2_b, int_lin_W, int_lin_b,
           lin1_W, lin1_b, lin2_W, lin2_b):
    row = edge_index[0].astype(jnp.int32)
    col = edge_index[1].astype(jnp.int32)
    pos128 = jnp.zeros((N_NODES, HID), jnp.float32).at[:, :3].set(pos)

    dpos = _dpos_call(pos128, row, col)
    wf_all = _wf_call(dpos, mlp_W1, mlp_b1, mlp_W2, mlp_b2)
    h, xh = _pre_call(x, emb_W, emb_b.reshape(1, HID), cf_lin1_W)

    zero = jnp.zeros((N_NODES, HID), jnp.float32)
    for l in range(NL):
        agg2 = _mp_call(l, xh, wf_all, row, col, zero)
        h, xh = _layer_call(l, agg2, h, cf_lin2_W, cf_lin2_b, int_lin_W,
                            int_lin_b, cf_lin1_W)

    ge, out = _head_call(h, lin1_W, lin1_b.reshape(1, HID), lin2_W,
                         lin2_b.reshape(1, N_CLASSES))
    return (ge, out)


# speedup vs baseline: 3.5441x; 1.0272x over previous
import functools

import numpy as np
import jax
import jax.numpy as jnp
from jax import lax
from jax.experimental import pallas as pl
from jax.experimental.pallas import tpu as pltpu
from jax.experimental.pallas import tpu_sc as plsc

N_NODES = 10000
N_EDGES = 320000
N_ATOMS = 21
N_CLASSES = 97
HID = 128
NG = 50
NL = 6
CUTOFF = 10.0

NC = 2
NS = 16
NW = NC * NS
CB = 64
NCHUNK = N_EDGES // CB
STRIPE = 624
TAIL = N_NODES - NS * STRIPE

_OFF = np.linspace(0.0, CUTOFF, NG).astype(np.float32)
_COEFF = np.float32(-0.5) / (_OFF[1] - _OFF[0]) ** 2
_LN2 = np.float32(np.log(2.0))

def _ssp(v):
    return (jnp.maximum(v, np.float32(0.0))
            + jnp.log(np.float32(1.0) + jnp.exp(-jnp.abs(v))) - _LN2)


def _sc_mesh():
    return plsc.VectorSubcoreMesh(core_axis_name="c", subcore_axis_name="s",
                                  num_cores=NC, num_subcores=NS)


def _dpos_body(pos_hbm, row_hbm, col_hbm, out_hbm, idx_r, idx_c, pr_v, pc_v, dd_v, sem):
    c = lax.axis_index("c")
    s = lax.axis_index("s")
    w = s * NC + c
    nk = (NCHUNK - w + NW - 1) // NW

    def chunk(i, carry):
        base = (w + i * NW) * CB
        pltpu.sync_copy(row_hbm.at[pl.ds(base, CB)], idx_r)
        pltpu.sync_copy(col_hbm.at[pl.ds(base, CB)], idx_c)
        pltpu.async_copy(pos_hbm.at[idx_r], pr_v, sem).wait()
        pltpu.async_copy(pos_hbm.at[idx_c], pc_v, sem).wait()

        def rowfn(r, cc):
            dd_v[r, :] = pr_v[r, :] - pc_v[r, :]
            return cc

        lax.fori_loop(0, CB, rowfn, 0)
        pltpu.sync_copy(dd_v, out_hbm.at[pl.ds(base, CB)])
        return carry

    lax.fori_loop(0, nk, chunk, 0)


@jax.jit
def _dpos_call(pos16, row, col):
    kfn = pl.kernel(
        _dpos_body,
        out_type=jax.ShapeDtypeStruct((N_EDGES, 16), jnp.float32),
        mesh=_sc_mesh(),
        compiler_params=pltpu.CompilerParams(use_tc_tiling_on_sc=False),
        scratch_types=[
            pltpu.VMEM((CB,), jnp.int32),
            pltpu.VMEM((CB,), jnp.int32),
            pltpu.VMEM((CB, 16), jnp.float32),
            pltpu.VMEM((CB, 16), jnp.float32),
            pltpu.VMEM((CB, 16), jnp.float32),
            pltpu.SemaphoreType.DMA,
        ],
    )
    return kfn(pos16, row, col)


NKFULL = 156
NPAIR = NKFULL // 3


def _mp_body(l, xh_hbm, wf_hbm, row_hbm, col_hbm, zero_hbm, out_hbm,
             idx_r, idx_c, rows, wfv, agg_sh,
             sg0, sg1, sg2, sw0, sw1, sw2, ss0, ss1, ss2):
    SG = (sg0, sg1, sg2)
    SW = (sw0, sw1, sw2)
    SS = (ss0, ss1, ss2)
    c = lax.axis_index("c")
    s = lax.axis_index("s")
    w = s * NC + c
    zbase = s * STRIPE
    pltpu.sync_copy(zero_hbm.at[pl.ds(zbase, STRIPE)],
                    agg_sh.at[pl.ds(zbase, STRIPE)])

    @pl.when(s == NS - 1)
    def _():
        pltpu.sync_copy(zero_hbm.at[pl.ds(NS * STRIPE, TAIL)],
                        agg_sh.at[pl.ds(NS * STRIPE, TAIL)])

    plsc.subcore_barrier()

    def start(t, b):
        base = (w + t * NW) * CB
        pltpu.sync_copy(row_hbm.at[pl.ds(base, CB)], idx_r.at[b])
        pltpu.sync_copy(col_hbm.at[pl.ds(base, CB)], idx_c.at[b])
        pltpu.async_copy(xh_hbm.at[idx_r.at[b]], rows.at[b], SG[b])
        pltpu.async_copy(wf_hbm.at[l, pl.ds(base, CB)], wfv.at[b], SW[b])

    def drain_scatter(b):
        pltpu.make_async_copy(rows.at[b], agg_sh.at[idx_c.at[b]], SS[b]).wait()

    def mul(b):
        def rowfn(r, cc):
            for j in range(HID // 16):
                sl = pl.ds(j * 16, 16)
                rows[b, r, sl] = rows[b, r, sl] * wfv[b, r, sl]
            return cc

        lax.fori_loop(0, CB, rowfn, 0)

    start(0, 0)
    start(1, 1)

    def block(j, r):
        t = 3 * j + r
        b2 = (r + 2) % 3
        pltpu.make_async_copy(xh_hbm.at[idx_r.at[r]], rows.at[r], SG[r]).wait()
        pltpu.make_async_copy(wf_hbm.at[l, pl.ds(0, CB)], wfv.at[r], SW[r]).wait()
        mul(r)
        pltpu.async_copy(rows.at[r], agg_sh.at[idx_c.at[r]], SS[r], add=True)
        return t, b2

    def pair(j, carry):
        t, b2 = block(j, 0)

        @pl.when(j >= 1)
        def _():
            drain_scatter(b2)

        start(t + 2, b2)

        t, b2 = block(j, 1)

        @pl.when(j < NPAIR - 1)
        def _():
            drain_scatter(b2)
            start(t + 2, b2)

        t, b2 = block(j, 2)

        @pl.when(j < NPAIR - 1)
        def _():
            drain_scatter(b2)
            start(t + 2, b2)

        return carry

    lax.fori_loop(0, NPAIR, pair, 0)
    drain_scatter(0)
    drain_scatter(1)
    drain_scatter(2)

    @pl.when(w < NCHUNK - NKFULL * NW)
    def _():
        base = (NKFULL * NW + w) * CB
        pltpu.sync_copy(row_hbm.at[pl.ds(base, CB)], idx_r.at[0])
        pltpu.sync_copy(col_hbm.at[pl.ds(base, CB)], idx_c.at[0])
        pltpu.async_copy(xh_hbm.at[idx_r.at[0]], rows.at[0], SG[0]).wait()
        pltpu.sync_copy(wf_hbm.at[l, pl.ds(base, CB)], wfv.at[0])
        mul(0)
        pltpu.sync_copy(rows.at[0], agg_sh.at[idx_c.at[0]], add=True)

    plsc.subcore_barrier()
    obase = c * N_NODES + s * STRIPE
    pltpu.sync_copy(agg_sh.at[pl.ds(s * STRIPE, STRIPE)],
                    out_hbm.at[pl.ds(obase, STRIPE)])

    @pl.when(s == NS - 1)
    def _():
        pltpu.sync_copy(agg_sh.at[pl.ds(NS * STRIPE, TAIL)],
                        out_hbm.at[pl.ds(c * N_NODES + NS * STRIPE, TAIL)])


@functools.partial(jax.jit, static_argnums=(0,))
def _mp_call(l, xh, wf_all, row, col, zero):
    kfn = pl.kernel(
        functools.partial(_mp_body, l),
        out_type=jax.ShapeDtypeStruct((2 * N_NODES, HID), jnp.float32),
        mesh=_sc_mesh(),
        scratch_types=[
            pltpu.VMEM((3, CB), jnp.int32),
            pltpu.VMEM((3, CB), jnp.int32),
            pltpu.VMEM((3, CB, HID), jnp.float32),
            pltpu.VMEM((3, CB, HID), jnp.float32),
            pltpu.VMEM_SHARED((N_NODES, HID), jnp.float32),
        ] + [pltpu.SemaphoreType.DMA] * 9,
    )
    return kfn(xh, wf_all, row, col, zero)


BE = 512


def _wf_body(dpos_ref, w1_ref, b1_ref, w2_ref, b2_ref, out_ref):
    dp = dpos_ref[...]
    d2 = jnp.sum(dp * dp, axis=1)
    wgt = jnp.sqrt(d2 + np.float32(1e-12))
    xx = wgt * np.float32(np.pi / CUTOFF)
    k = jnp.floor(xx * np.float32(1.0 / np.pi) + np.float32(0.5))
    r = xx - k * np.float32(np.pi)
    r2 = r * r
    cpoly = np.float32(-2.605210867e-07)
    for coef in (2.479886187e-05, -1.388829677e-03, 4.166645418e-02,
                 -4.999999389e-01, 9.999999724e-01):
        cpoly = cpoly * r2 + np.float32(coef)
    kodd = k - np.float32(2.0) * jnp.floor(k * np.float32(0.5))
    sgn = np.float32(1.0) - np.float32(2.0) * kodd
    cutc = np.float32(0.5) * (sgn * cpoly + np.float32(1.0))
    off = (lax.broadcasted_iota(jnp.int32, (1, NG), 1).astype(jnp.float32)
           * np.float32(CUTOFF / (NG - 1)))
    diff = wgt[:, None] - off
    ea = jnp.exp(_COEFF * diff * diff)
    for l in range(NL):
        t = jnp.dot(ea, w1_ref[l], preferred_element_type=jnp.float32) + b1_ref[l]
        t = _ssp(t)
        wf = jnp.dot(t, w2_ref[l], preferred_element_type=jnp.float32) + b2_ref[l]
        out_ref[l] = wf * cutc[:, None]


@jax.jit
def _wf_call(dpos, mlp_W1, mlp_b1, mlp_W2, mlp_b2):
    grid = (N_EDGES // BE,)
    return pl.pallas_call(
        _wf_body,
        grid=grid,
        in_specs=[
            pl.BlockSpec((BE, 16), lambda i: (i, 0)),
            pl.BlockSpec((NL, NG, HID), lambda i: (0, 0, 0)),
            pl.BlockSpec((NL, HID), lambda i: (0, 0)),
            pl.BlockSpec((NL, HID, HID), lambda i: (0, 0, 0)),
            pl.BlockSpec((NL, HID), lambda i: (0, 0)),
        ],
        out_specs=pl.BlockSpec((NL, BE, HID), lambda i: (0, i, 0)),
        out_shape=jax.ShapeDtypeStruct((NL, N_EDGES, HID), jnp.float32),
    )(dpos, mlp_W1, mlp_b1, mlp_W2, mlp_b2)


BN = 1000


def _pre_body(x_ref, ew_ref, eb_ref, cf1_ref, h_ref, xh_ref):
    h0 = jnp.dot(x_ref[...], ew_ref[...], preferred_element_type=jnp.float32) + eb_ref[...]
    h_ref[...] = h0
    xh_ref[...] = jnp.dot(h0, cf1_ref[0], preferred_element_type=jnp.float32)


@jax.jit
def _pre_call(x, emb_W, emb_b2d, cf_lin1_W):
    grid = (N_NODES // BN,)
    return pl.pallas_call(
        _pre_body,
        grid=grid,
        in_specs=[
            pl.BlockSpec((BN, N_ATOMS), lambda i: (i, 0)),
            pl.BlockSpec((N_ATOMS, HID), lambda i: (0, 0)),
            pl.BlockSpec((1, HID), lambda i: (0, 0)),
            pl.BlockSpec((NL, HID, HID), lambda i: (0, 0, 0)),
        ],
        out_specs=[
            pl.BlockSpec((BN, HID), lambda i: (i, 0)),
            pl.BlockSpec((BN, HID), lambda i: (i, 0)),
        ],
        out_shape=[
            jax.ShapeDtypeStruct((N_NODES, HID), jnp.float32),
            jax.ShapeDtypeStruct((N_NODES, HID), jnp.float32),
        ],
    )(x, emb_W, emb_b2d, cf_lin1_W)


def _layer_body(l, a0_ref, a1_ref, h_ref, cf2w_ref, cf2b_ref, intw_ref, intb_ref,
                cf1_ref, hout_ref, xhout_ref):
    agg = a0_ref[...] + a1_ref[...]
    t = jnp.dot(agg, cf2w_ref[l], preferred_element_type=jnp.float32) + cf2b_ref[l]
    t = _ssp(t)
    t = jnp.dot(t, intw_ref[l], preferred_element_type=jnp.float32) + intb_ref[l]
    hn = h_ref[...] + t
    hout_ref[...] = hn
    xhout_ref[...] = jnp.dot(hn, cf1_ref[(l + 1) % NL], preferred_element_type=jnp.float32)


@functools.partial(jax.jit, static_argnums=(0,))
def _layer_call(l, agg2, h, cf_lin2_W, cf_lin2_b, int_lin_W, int_lin_b, cf_lin1_W):
    grid = (N_NODES // BN,)
    nb = N_NODES // BN
    return pl.pallas_call(
        functools.partial(_layer_body, l),
        grid=grid,
        in_specs=[
            pl.BlockSpec((BN, HID), lambda i: (i, 0)),
            pl.BlockSpec((BN, HID), lambda i, _nb=nb: (i + _nb, 0)),
            pl.BlockSpec((BN, HID), lambda i: (i, 0)),
            pl.BlockSpec((NL, HID, HID), lambda i: (0, 0, 0)),
            pl.BlockSpec((NL, HID), lambda i: (0, 0)),
            pl.BlockSpec((NL, HID, HID), lambda i: (0, 0, 0)),
            pl.BlockSpec((NL, HID), lambda i: (0, 0)),
            pl.BlockSpec((NL, HID, HID), lambda i: (0, 0, 0)),
        ],
        out_specs=[
            pl.BlockSpec((BN, HID), lambda i: (i, 0)),
            pl.BlockSpec((BN, HID), lambda i: (i, 0)),
        ],
        out_shape=[
            jax.ShapeDtypeStruct((N_NODES, HID), jnp.float32),
            jax.ShapeDtypeStruct((N_NODES, HID), jnp.float32),
        ],
    )(agg2, agg2, h, cf_lin2_W, cf_lin2_b, int_lin_W, int_lin_b, cf_lin1_W)


def _head_body(h_ref, l1w_ref, l1b_ref, l2w_ref, l2b_ref, ge_ref, out_ref, acc_ref):
    i = pl.program_id(0)
    hf = jnp.maximum(
        jnp.dot(h_ref[...], l1w_ref[...], preferred_element_type=jnp.float32)
        + l1b_ref[...], np.float32(0.0))
    part = jnp.sum(hf, axis=0, keepdims=True)

    @pl.when(i == 0)
    def _():
        acc_ref[...] = part

    @pl.when(i > 0)
    def _():
        acc_ref[...] = acc_ref[...] + part

    @pl.when(i == (N_NODES // BN) - 1)
    def _():
        ge = acc_ref[...] / np.float32(N_NODES)
        ge_ref[...] = ge
        h2 = jnp.maximum(
            jnp.dot(ge, l1w_ref[...], preferred_element_type=jnp.float32)
            + l1b_ref[...], np.float32(0.0))
        out_ref[...] = (jnp.dot(h2, l2w_ref[...], preferred_element_type=jnp.float32)
                        + l2b_ref[...])


@jax.jit
def _head_call(h, lin1_W, lin1_b2d, lin2_W, lin2_b2d):
    grid = (N_NODES // BN,)
    return pl.pallas_call(
        _head_body,
        grid=grid,
        in_specs=[
            pl.BlockSpec((BN, HID), lambda i: (i, 0)),
            pl.BlockSpec((HID, HID), lambda i: (0, 0)),
            pl.BlockSpec((1, HID), lambda i: (0, 0)),
            pl.BlockSpec((HID, N_CLASSES), lambda i: (0, 0)),
            pl.BlockSpec((1, N_CLASSES), lambda i: (0, 0)),
        ],
        out_specs=[
            pl.BlockSpec((1, HID), lambda i: (0, 0)),
            pl.BlockSpec((1, N_CLASSES), lambda i: (0, 0)),
        ],
        out_shape=[
            jax.ShapeDtypeStruct((1, HID), jnp.float32),
            jax.ShapeDtypeStruct((1, N_CLASSES), jnp.float32),
        ],
        scratch_shapes=[pltpu.VMEM((1, HID), jnp.float32)],
    )(h, lin1_W, lin1_b2d, lin2_W, lin2_b2d)


def kernel(x, pos, edge_index, batch, emb_W, emb_b, mlp_W1, mlp_b1, mlp_W2, mlp_b2,
           cf_lin1_W, cf_lin2_W, cf_lin2_b, int_lin_W, int_lin_b,
           lin1_W, lin1_b, lin2_W, lin2_b):
    row = edge_index[0].astype(jnp.int32)
    col = edge_index[1].astype(jnp.int32)
    pos16 = jnp.zeros((N_NODES, 16), jnp.float32).at[:, :3].set(pos)

    dpos = _dpos_call(pos16, row, col)
    wf_all = _wf_call(dpos, mlp_W1, mlp_b1, mlp_W2, mlp_b2)
    h, xh = _pre_call(x, emb_W, emb_b.reshape(1, HID), cf_lin1_W)

    zero = jnp.zeros((N_NODES, HID), jnp.float32)
    for l in range(NL):
        agg2 = _mp_call(l, xh, wf_all, row, col, zero)
        h, xh = _layer_call(l, agg2, h, cf_lin2_W, cf_lin2_b, int_lin_W,
                            int_lin_b, cf_lin1_W)

    ge, out = _head_call(h, lin1_W, lin1_b.reshape(1, HID), lin2_W,
                         lin2_b.reshape(1, N_CLASSES))
    return (ge, out)


# speedup vs baseline: 3.7784x; 1.0661x over previous
import functools

import numpy as np
import jax
import jax.numpy as jnp
from jax import lax
from jax.experimental import pallas as pl
from jax.experimental.pallas import tpu as pltpu
from jax.experimental.pallas import tpu_sc as plsc

N_NODES = 10000
N_EDGES = 320000
N_ATOMS = 21
N_CLASSES = 97
HID = 128
NG = 50
NL = 6
CUTOFF = 10.0

NC = 2
NS = 16
NW = NC * NS
CB = 64
NCHUNK = N_EDGES // CB
STRIPE = 624
TAIL = N_NODES - NS * STRIPE

_OFF = np.linspace(0.0, CUTOFF, NG).astype(np.float32)
_COEFF = np.float32(-0.5) / (_OFF[1] - _OFF[0]) ** 2
_LN2 = np.float32(np.log(2.0))

def _ssp(v):
    return (jnp.maximum(v, np.float32(0.0))
            + jnp.log(np.float32(1.0) + jnp.exp(-jnp.abs(v))) - _LN2)


def _sc_mesh():
    return plsc.VectorSubcoreMesh(core_axis_name="c", subcore_axis_name="s",
                                  num_cores=NC, num_subcores=NS)


def _dpos_body(pos_hbm, row_hbm, col_hbm, out_hbm, idx_r, idx_c, pr, pc,
               sr0, sr1, sr2, sc0, sc1, sc2):
    SR = (sr0, sr1, sr2)
    SC = (sc0, sc1, sc2)
    c = lax.axis_index("c")
    s = lax.axis_index("s")
    w = s * NC + c

    def start(t, b):
        base = (w + t * NW) * CB
        pltpu.sync_copy(row_hbm.at[pl.ds(base, CB)], idx_r.at[b])
        pltpu.sync_copy(col_hbm.at[pl.ds(base, CB)], idx_c.at[b])
        pltpu.async_copy(pos_hbm.at[idx_r.at[b]], pr.at[b], SR[b])
        pltpu.async_copy(pos_hbm.at[idx_c.at[b]], pc.at[b], SC[b])

    def block(j, r):
        t = 3 * j + r
        base = (w + t * NW) * CB
        pltpu.make_async_copy(pos_hbm.at[idx_r.at[r]], pr.at[r], SR[r]).wait()
        pltpu.make_async_copy(pos_hbm.at[idx_c.at[r]], pc.at[r], SC[r]).wait()

        def rowfn(rr, cc):
            pr[r, rr, :] = pr[r, rr, :] - pc[r, rr, :]
            return cc

        lax.fori_loop(0, CB, rowfn, 0)
        pltpu.sync_copy(pr.at[r], out_hbm.at[pl.ds(base, CB)])
        return t

    start(0, 0)
    start(1, 1)

    def triple(j, carry):
        t = block(j, 0)
        start(t + 2, (0 + 2) % 3)
        t = block(j, 1)

        @pl.when(j < NPAIR - 1)
        def _():
            start(t + 2, (1 + 2) % 3)

        t = block(j, 2)

        @pl.when(j < NPAIR - 1)
        def _():
            start(t + 2, (2 + 2) % 3)

        return carry

    lax.fori_loop(0, NPAIR, triple, 0)

    @pl.when(w < NCHUNK - NKFULL * NW)
    def _():
        base = (NKFULL * NW + w) * CB
        pltpu.sync_copy(row_hbm.at[pl.ds(base, CB)], idx_r.at[0])
        pltpu.sync_copy(col_hbm.at[pl.ds(base, CB)], idx_c.at[0])
        pltpu.async_copy(pos_hbm.at[idx_r.at[0]], pr.at[0], SR[0]).wait()
        pltpu.async_copy(pos_hbm.at[idx_c.at[0]], pc.at[0], SC[0]).wait()

        def rowfn(rr, cc):
            pr[0, rr, :] = pr[0, rr, :] - pc[0, rr, :]
            return cc

        lax.fori_loop(0, CB, rowfn, 0)
        pltpu.sync_copy(pr.at[0], out_hbm.at[pl.ds(base, CB)])


@jax.jit
def _dpos_call(pos16, row, col):
    kfn = pl.kernel(
        _dpos_body,
        out_type=jax.ShapeDtypeStruct((N_EDGES, 16), jnp.float32),
        mesh=_sc_mesh(),
        compiler_params=pltpu.CompilerParams(use_tc_tiling_on_sc=False),
        scratch_types=[
            pltpu.VMEM((3, CB), jnp.int32),
            pltpu.VMEM((3, CB), jnp.int32),
            pltpu.VMEM((3, CB, 16), jnp.float32),
            pltpu.VMEM((3, CB, 16), jnp.float32),
        ] + [pltpu.SemaphoreType.DMA] * 6,
    )
    return kfn(pos16, row, col)


NKFULL = 156
NPAIR = NKFULL // 3


def _mp_body(l, xh_hbm, wf_hbm, row_hbm, col_hbm, zero_hbm, out_hbm,
             idx_r, idx_c, rows, wfv, agg_sh,
             sg0, sg1, sg2, sw0, sw1, sw2, ss0, ss1, ss2):
    SG = (sg0, sg1, sg2)
    SW = (sw0, sw1, sw2)
    SS = (ss0, ss1, ss2)
    c = lax.axis_index("c")
    s = lax.axis_index("s")
    w = s * NC + c
    zbase = s * STRIPE
    pltpu.sync_copy(zero_hbm.at[pl.ds(zbase, STRIPE)],
                    agg_sh.at[pl.ds(zbase, STRIPE)])

    @pl.when(s == NS - 1)
    def _():
        pltpu.sync_copy(zero_hbm.at[pl.ds(NS * STRIPE, TAIL)],
                        agg_sh.at[pl.ds(NS * STRIPE, TAIL)])

    plsc.subcore_barrier()

    def start(t, b):
        base = (w + t * NW) * CB
        pltpu.sync_copy(row_hbm.at[pl.ds(base, CB)], idx_r.at[b])
        pltpu.sync_copy(col_hbm.at[pl.ds(base, CB)], idx_c.at[b])
        pltpu.async_copy(xh_hbm.at[idx_r.at[b]], rows.at[b], SG[b])
        pltpu.async_copy(wf_hbm.at[l, pl.ds(base, CB)], wfv.at[b], SW[b])

    def drain_scatter(b):
        pltpu.make_async_copy(rows.at[b], agg_sh.at[idx_c.at[b]], SS[b]).wait()

    def mul(b):
        def rowfn(r, cc):
            for j in range(HID // 16):
                sl = pl.ds(j * 16, 16)
                rows[b, r, sl] = rows[b, r, sl] * wfv[b, r, sl]
            return cc

        lax.fori_loop(0, CB, rowfn, 0)

    start(0, 0)
    start(1, 1)

    def block(j, r):
        t = 3 * j + r
        b2 = (r + 2) % 3
        pltpu.make_async_copy(xh_hbm.at[idx_r.at[r]], rows.at[r], SG[r]).wait()
        pltpu.make_async_copy(wf_hbm.at[l, pl.ds(0, CB)], wfv.at[r], SW[r]).wait()
        mul(r)
        pltpu.async_copy(rows.at[r], agg_sh.at[idx_c.at[r]], SS[r], add=True)
        return t, b2

    def pair(j, carry):
        t, b2 = block(j, 0)

        @pl.when(j >= 1)
        def _():
            drain_scatter(b2)

        start(t + 2, b2)

        t, b2 = block(j, 1)

        @pl.when(j < NPAIR - 1)
        def _():
            drain_scatter(b2)
            start(t + 2, b2)

        t, b2 = block(j, 2)

        @pl.when(j < NPAIR - 1)
        def _():
            drain_scatter(b2)
            start(t + 2, b2)

        return carry

    lax.fori_loop(0, NPAIR, pair, 0)
    drain_scatter(0)
    drain_scatter(1)
    drain_scatter(2)

    @pl.when(w < NCHUNK - NKFULL * NW)
    def _():
        base = (NKFULL * NW + w) * CB
        pltpu.sync_copy(row_hbm.at[pl.ds(base, CB)], idx_r.at[0])
        pltpu.sync_copy(col_hbm.at[pl.ds(base, CB)], idx_c.at[0])
        pltpu.async_copy(xh_hbm.at[idx_r.at[0]], rows.at[0], SG[0]).wait()
        pltpu.sync_copy(wf_hbm.at[l, pl.ds(base, CB)], wfv.at[0])
        mul(0)
        pltpu.sync_copy(rows.at[0], agg_sh.at[idx_c.at[0]], add=True)

    plsc.subcore_barrier()
    obase = c * N_NODES + s * STRIPE
    pltpu.sync_copy(agg_sh.at[pl.ds(s * STRIPE, STRIPE)],
                    out_hbm.at[pl.ds(obase, STRIPE)])

    @pl.when(s == NS - 1)
    def _():
        pltpu.sync_copy(agg_sh.at[pl.ds(NS * STRIPE, TAIL)],
                        out_hbm.at[pl.ds(c * N_NODES + NS * STRIPE, TAIL)])


@functools.partial(jax.jit, static_argnums=(0,))
def _mp_call(l, xh, wf_all, row, col, zero):
    kfn = pl.kernel(
        functools.partial(_mp_body, l),
        out_type=jax.ShapeDtypeStruct((2 * N_NODES, HID), jnp.float32),
        mesh=_sc_mesh(),
        scratch_types=[
            pltpu.VMEM((3, CB), jnp.int32),
            pltpu.VMEM((3, CB), jnp.int32),
            pltpu.VMEM((3, CB, HID), jnp.float32),
            pltpu.VMEM((3, CB, HID), jnp.float32),
            pltpu.VMEM_SHARED((N_NODES, HID), jnp.float32),
        ] + [pltpu.SemaphoreType.DMA] * 9,
    )
    return kfn(xh, wf_all, row, col, zero)


BE = 512


def _wf_body(dpos_ref, w1_ref, b1_ref, w2_ref, b2_ref, out_ref):
    dp = dpos_ref[...]
    d2 = jnp.sum(dp * dp, axis=1)
    wgt = jnp.sqrt(d2 + np.float32(1e-12))
    xx = wgt * np.float32(np.pi / CUTOFF)
    k = jnp.floor(xx * np.float32(1.0 / np.pi) + np.float32(0.5))
    r = xx - k * np.float32(np.pi)
    r2 = r * r
    cpoly = np.float32(-2.605210867e-07)
    for coef in (2.479886187e-05, -1.388829677e-03, 4.166645418e-02,
                 -4.999999389e-01, 9.999999724e-01):
        cpoly = cpoly * r2 + np.float32(coef)
    kodd = k - np.float32(2.0) * jnp.floor(k * np.float32(0.5))
    sgn = np.float32(1.0) - np.float32(2.0) * kodd
    cutc = np.float32(0.5) * (sgn * cpoly + np.float32(1.0))
    off = (lax.broadcasted_iota(jnp.int32, (1, NG), 1).astype(jnp.float32)
           * np.float32(CUTOFF / (NG - 1)))
    diff = wgt[:, None] - off
    ea = jnp.exp(_COEFF * diff * diff)
    for l in range(NL):
        t = jnp.dot(ea, w1_ref[l], preferred_element_type=jnp.float32) + b1_ref[l]
        t = _ssp(t)
        wf = jnp.dot(t, w2_ref[l], preferred_element_type=jnp.float32) + b2_ref[l]
        out_ref[l] = wf * cutc[:, None]


@jax.jit
def _wf_call(dpos, mlp_W1, mlp_b1, mlp_W2, mlp_b2):
    grid = (N_EDGES // BE,)
    return pl.pallas_call(
        _wf_body,
        grid=grid,
        in_specs=[
            pl.BlockSpec((BE, 16), lambda i: (i, 0)),
            pl.BlockSpec((NL, NG, HID), lambda i: (0, 0, 0)),
            pl.BlockSpec((NL, HID), lambda i: (0, 0)),
            pl.BlockSpec((NL, HID, HID), lambda i: (0, 0, 0)),
            pl.BlockSpec((NL, HID), lambda i: (0, 0)),
        ],
        out_specs=pl.BlockSpec((NL, BE, HID), lambda i: (0, i, 0)),
        out_shape=jax.ShapeDtypeStruct((NL, N_EDGES, HID), jnp.float32),
    )(dpos, mlp_W1, mlp_b1, mlp_W2, mlp_b2)


BN = 1000


def _pre_body(x_ref, ew_ref, eb_ref, cf1_ref, h_ref, xh_ref):
    h0 = jnp.dot(x_ref[...], ew_ref[...], preferred_element_type=jnp.float32) + eb_ref[...]
    h_ref[...] = h0
    xh_ref[...] = jnp.dot(h0, cf1_ref[0], preferred_element_type=jnp.float32)


@jax.jit
def _pre_call(x, emb_W, emb_b2d, cf_lin1_W):
    grid = (N_NODES // BN,)
    return pl.pallas_call(
        _pre_body,
        grid=grid,
        in_specs=[
            pl.BlockSpec((BN, N_ATOMS), lambda i: (i, 0)),
            pl.BlockSpec((N_ATOMS, HID), lambda i: (0, 0)),
            pl.BlockSpec((1, HID), lambda i: (0, 0)),
            pl.BlockSpec((NL, HID, HID), lambda i: (0, 0, 0)),
        ],
        out_specs=[
            pl.BlockSpec((BN, HID), lambda i: (i, 0)),
            pl.BlockSpec((BN, HID), lambda i: (i, 0)),
        ],
        out_shape=[
            jax.ShapeDtypeStruct((N_NODES, HID), jnp.float32),
            jax.ShapeDtypeStruct((N_NODES, HID), jnp.float32),
        ],
    )(x, emb_W, emb_b2d, cf_lin1_W)


def _layer_body(l, a0_ref, a1_ref, h_ref, cf2w_ref, cf2b_ref, intw_ref, intb_ref,
                cf1_ref, hout_ref, xhout_ref):
    agg = a0_ref[...] + a1_ref[...]
    t = jnp.dot(agg, cf2w_ref[l], preferred_element_type=jnp.float32) + cf2b_ref[l]
    t = _ssp(t)
    t = jnp.dot(t, intw_ref[l], preferred_element_type=jnp.float32) + intb_ref[l]
    hn = h_ref[...] + t
    hout_ref[...] = hn
    xhout_ref[...] = jnp.dot(hn, cf1_ref[(l + 1) % NL], preferred_element_type=jnp.float32)


@functools.partial(jax.jit, static_argnums=(0,))
def _layer_call(l, agg2, h, cf_lin2_W, cf_lin2_b, int_lin_W, int_lin_b, cf_lin1_W):
    grid = (N_NODES // BN,)
    nb = N_NODES // BN
    return pl.pallas_call(
        functools.partial(_layer_body, l),
        grid=grid,
        in_specs=[
            pl.BlockSpec((BN, HID), lambda i: (i, 0)),
            pl.BlockSpec((BN, HID), lambda i, _nb=nb: (i + _nb, 0)),
            pl.BlockSpec((BN, HID), lambda i: (i, 0)),
            pl.BlockSpec((NL, HID, HID), lambda i: (0, 0, 0)),
            pl.BlockSpec((NL, HID), lambda i: (0, 0)),
            pl.BlockSpec((NL, HID, HID), lambda i: (0, 0, 0)),
            pl.BlockSpec((NL, HID), lambda i: (0, 0)),
            pl.BlockSpec((NL, HID, HID), lambda i: (0, 0, 0)),
        ],
        out_specs=[
            pl.BlockSpec((BN, HID), lambda i: (i, 0)),
            pl.BlockSpec((BN, HID), lambda i: (i, 0)),
        ],
        out_shape=[
            jax.ShapeDtypeStruct((N_NODES, HID), jnp.float32),
            jax.ShapeDtypeStruct((N_NODES, HID), jnp.float32),
        ],
    )(agg2, agg2, h, cf_lin2_W, cf_lin2_b, int_lin_W, int_lin_b, cf_lin1_W)


def _head_body(h_ref, l1w_ref, l1b_ref, l2w_ref, l2b_ref, ge_ref, out_ref, acc_ref):
    i = pl.program_id(0)
    hf = jnp.maximum(
        jnp.dot(h_ref[...], l1w_ref[...], preferred_element_type=jnp.float32)
        + l1b_ref[...], np.float32(0.0))
    part = jnp.sum(hf, axis=0, keepdims=True)

    @pl.when(i == 0)
    def _():
        acc_ref[...] = part

    @pl.when(i > 0)
    def _():
        acc_ref[...] = acc_ref[...] + part

    @pl.when(i == (N_NODES // BN) - 1)
    def _():
        ge = acc_ref[...] / np.float32(N_NODES)
        ge_ref[...] = ge
        h2 = jnp.maximum(
            jnp.dot(ge, l1w_ref[...], preferred_element_type=jnp.float32)
            + l1b_ref[...], np.float32(0.0))
        out_ref[...] = (jnp.dot(h2, l2w_ref[...], preferred_element_type=jnp.float32)
                        + l2b_ref[...])


@jax.jit
def _head_call(h, lin1_W, lin1_b2d, lin2_W, lin2_b2d):
    grid = (N_NODES // BN,)
    return pl.pallas_call(
        _head_body,
        grid=grid,
        in_specs=[
            pl.BlockSpec((BN, HID), lambda i: (i, 0)),
            pl.BlockSpec((HID, HID), lambda i: (0, 0)),
            pl.BlockSpec((1, HID), lambda i: (0, 0)),
            pl.BlockSpec((HID, N_CLASSES), lambda i: (0, 0)),
            pl.BlockSpec((1, N_CLASSES), lambda i: (0, 0)),
        ],
        out_specs=[
            pl.BlockSpec((1, HID), lambda i: (0, 0)),
            pl.BlockSpec((1, N_CLASSES), lambda i: (0, 0)),
        ],
        out_shape=[
            jax.ShapeDtypeStruct((1, HID), jnp.float32),
            jax.ShapeDtypeStruct((1, N_CLASSES), jnp.float32),
        ],
        scratch_shapes=[pltpu.VMEM((1, HID), jnp.float32)],
    )(h, lin1_W, lin1_b2d, lin2_W, lin2_b2d)


def kernel(x, pos, edge_index, batch, emb_W, emb_b, mlp_W1, mlp_b1, mlp_W2, mlp_b2,
           cf_lin1_W, cf_lin2_W, cf_lin2_b, int_lin_W, int_lin_b,
           lin1_W, lin1_b, lin2_W, lin2_b):
    row = edge_index[0].astype(jnp.int32)
    col = edge_index[1].astype(jnp.int32)
    pos16 = jnp.zeros((N_NODES, 16), jnp.float32).at[:, :3].set(pos)

    dpos = _dpos_call(pos16, row, col)
    wf_all = _wf_call(dpos, mlp_W1, mlp_b1, mlp_W2, mlp_b2)
    h, xh = _pre_call(x, emb_W, emb_b.reshape(1, HID), cf_lin1_W)

    zero = jnp.zeros((N_NODES, HID), jnp.float32)
    for l in range(NL):
        agg2 = _mp_call(l, xh, wf_all, row, col, zero)
        h, xh = _layer_call(l, agg2, h, cf_lin2_W, cf_lin2_b, int_lin_W,
                            int_lin_b, cf_lin1_W)

    ge, out = _head_call(h, lin1_W, lin1_b.reshape(1, HID), lin2_W,
                         lin2_b.reshape(1, N_CLASSES))
    return (ge, out)
